# baseline (speedup 1.0000x reference)
import functools

import jax
import jax.numpy as jnp
from jax.experimental import pallas as pl
from jax.experimental.pallas import tpu as pltpu
from jax.experimental.pallas import tpu_sc as plsc

_NUM_OPS = 64

_mesh = plsc.ScalarSubcoreMesh(axis_name="c", num_cores=1)


@functools.partial(
    pl.kernel,
    out_type=(
        jax.ShapeDtypeStruct((1,), jnp.float32),
        jax.ShapeDtypeStruct((1,), jnp.int32),
    ),
    mesh=_mesh,
    scratch_types=[
        pltpu.SMEM((_NUM_OPS,), jnp.float32),
        pltpu.SMEM((1,), jnp.float32),
        pltpu.SMEM((1,), jnp.int32),
        pltpu.SemaphoreType.DMA,
        pltpu.SemaphoreType.DMA,
        pltpu.SemaphoreType.DMA,
    ],
)
def _router(op_hbm, w_hbm, idx_hbm, x_s, w_s, i_s, sem_x, sem_w, sem_i):
    cp_x = pltpu.async_copy(op_hbm, x_s, sem_x)
    w_s[0] = jnp.float32(1.0)
    cp_w = pltpu.async_copy(w_s, w_hbm, sem_w)
    cp_x.wait()
    _C = 4
    _SEG = _NUM_OPS // _C
    ms, gs = [], []
    for c in range(_C):
        m = x_s[c * _SEG]
        g = jnp.int32(c * _SEG)
        for l in range(c * _SEG + 1, (c + 1) * _SEG):
            v_l = x_s[l]
            take = v_l > m
            m = jnp.where(take, v_l, m)
            g = jnp.where(take, jnp.int32(l), g)
        ms.append(m)
        gs.append(g)
    m, gidx = ms[0], gs[0]
    for c in range(1, _C):
        take = ms[c] > m
        m = jnp.where(take, ms[c], m)
        gidx = jnp.where(take, gs[c], gidx)
    i_s[0] = gidx
    cp_i = pltpu.async_copy(i_s, idx_hbm, sem_i)
    cp_w.wait()
    cp_i.wait()


def kernel(op_enc):
    return _router(op_enc)
